# baseline (speedup 1.0000x reference)
import functools
import math

import jax
import jax.numpy as jnp
from jax import lax
from jax.experimental import pallas as pl
from jax.experimental.pallas import tpu as pltpu
from jax.experimental.pallas import tpu_sc as plsc

B = 16384
ED = 16
NUM_TABLES = 5
INPUT_DIM = 4 + NUM_TABLES * ED
HIDDEN = int(math.ceil((INPUT_DIM + 1) * 0.67))
OUT_DIM = 2

NC = 2
NS = 16
NW = NC * NS
BPW = B // NW
CHUNK = 128
NCH = BPW // CHUNK
NIR = NUM_TABLES * NCH

BLK = 2048


def _sc_gather(bin_t, bout_t, pin_t, pout_t, proto_t, idx):
    mesh = plsc.VectorSubcoreMesh(
        core_axis_name="c", subcore_axis_name="s", num_cores=NC, num_subcores=NS
    )

    @functools.partial(
        pl.kernel,
        out_type=jax.ShapeDtypeStruct((NUM_TABLES, B, ED), jnp.float32),
        mesh=mesh,
        scratch_types=[
            pltpu.VMEM((NIR, CHUNK), jnp.int32),
            pltpu.VMEM((NUM_TABLES, BPW, ED), jnp.float32),
            pltpu.SemaphoreType.DMA,
        ],
        compiler_params=pltpu.CompilerParams(
            use_tc_tiling_on_sc=False, needs_layout_passes=False
        ),
    )
    def k(bin_h, bout_h, pin_h, pout_h, proto_h, idx_h, out_h, idx_v,
          rows_v, sem):
        wid = lax.axis_index("s") * NC + lax.axis_index("c")
        base = wid * BPW
        pltpu.sync_copy(idx_h.at[pl.ds(wid * NIR, NIR)], idx_v)
        tables = (bin_h, bout_h, pin_h, pout_h, proto_h)
        copies = []
        for j, tab in enumerate(tables):
            for c in range(NCH):
                copies.append(
                    pltpu.async_copy(
                        tab.at[idx_v.at[j * NCH + c]],
                        rows_v.at[j, pl.ds(c * CHUNK, CHUNK)],
                        sem,
                    )
                )
        for cp in copies:
            cp.wait()
        for j in range(NUM_TABLES):
            pltpu.sync_copy(rows_v.at[j], out_h.at[j, pl.ds(base, BPW)])

    return k(bin_t, bout_t, pin_t, pout_t, proto_t, idx)


def _tc_mlp(x, e, W1x, W1e, b1, W2, b2, W3, b3):
    def body(x_ref, e_ref, w1x_ref, w1e_ref, b1_ref, w2_ref, b2_ref, w3_ref,
             b3_ref, o_ref):
        h = jnp.dot(x_ref[:], w1x_ref[:], preferred_element_type=jnp.float32)
        for j in range(NUM_TABLES):
            h = h + jnp.dot(e_ref[j], w1e_ref[j],
                            preferred_element_type=jnp.float32)
        h = jnp.maximum(h + b1_ref[:], 0.0)
        h = jnp.maximum(
            jnp.dot(h, w2_ref[:], preferred_element_type=jnp.float32) + b2_ref[:],
            0.0,
        )
        o_ref[:] = (
            jnp.dot(h, w3_ref[:], preferred_element_type=jnp.float32) + b3_ref[:]
        )

    return pl.pallas_call(
        body,
        grid=(B // BLK,),
        in_specs=[
            pl.BlockSpec((BLK, 9), lambda i: (i, 0)),
            pl.BlockSpec((NUM_TABLES, BLK, ED), lambda i: (0, i, 0)),
            pl.BlockSpec((9, HIDDEN), lambda i: (0, 0)),
            pl.BlockSpec((NUM_TABLES, ED, HIDDEN), lambda i: (0, 0, 0)),
            pl.BlockSpec((1, HIDDEN), lambda i: (0, 0)),
            pl.BlockSpec((HIDDEN, HIDDEN), lambda i: (0, 0)),
            pl.BlockSpec((1, HIDDEN), lambda i: (0, 0)),
            pl.BlockSpec((HIDDEN, OUT_DIM), lambda i: (0, 0)),
            pl.BlockSpec((1, OUT_DIM), lambda i: (0, 0)),
        ],
        out_specs=pl.BlockSpec((BLK, OUT_DIM), lambda i: (i, 0)),
        out_shape=jax.ShapeDtypeStruct((B, OUT_DIM), jnp.float32),
    )(x, e, W1x, W1e, b1, W2, b2, W3, b3)


def kernel(x, bin_table, bout_table, pin_table, pout_table, proto_table,
           W1, b1, W2, b2, W3, b3):
    idx = (
        x[:, 4:9]
        .astype(jnp.int32)
        .T.reshape(NUM_TABLES, NW, NCH, CHUNK)
        .transpose(1, 0, 2, 3)
        .reshape(NW * NIR, CHUNK)
    )
    e = _sc_gather(bin_table, bout_table, pin_table, pout_table, proto_table,
                   idx)
    W1x = jnp.concatenate(
        [W1[0:4], jnp.zeros((5, HIDDEN), W1.dtype)], axis=0
    )
    W1e = W1[4:].reshape(NUM_TABLES, ED, HIDDEN)
    return _tc_mlp(x, e, W1x, W1e, b1.reshape(1, -1), W2, b2.reshape(1, -1),
                   W3, b3.reshape(1, -1))


# speedup vs baseline: 1.0862x; 1.0862x over previous
import functools
import math

import jax
import jax.numpy as jnp
from jax import lax
from jax.experimental import pallas as pl
from jax.experimental.pallas import tpu as pltpu
from jax.experimental.pallas import tpu_sc as plsc

B = 16384
ED = 16
NUM_TABLES = 5
INPUT_DIM = 4 + NUM_TABLES * ED
HIDDEN = int(math.ceil((INPUT_DIM + 1) * 0.67))
OUT_DIM = 2

NC = 2
NS = 16
NW = NC * NS
BPW = B // NW
CHUNK = 128
NCH = BPW // CHUNK
NIR = NUM_TABLES * NCH

G = 8
BG = B // G
GBLK = 256
H8 = G * HIDDEN
O8 = G * OUT_DIM


def _sc_gather(bin_t, bout_t, pin_t, pout_t, proto_t, idx):
    mesh = plsc.VectorSubcoreMesh(
        core_axis_name="c", subcore_axis_name="s", num_cores=NC, num_subcores=NS
    )

    @functools.partial(
        pl.kernel,
        out_type=jax.ShapeDtypeStruct((NUM_TABLES, B, ED), jnp.float32),
        mesh=mesh,
        scratch_types=[
            pltpu.VMEM((NIR, CHUNK), jnp.int32),
            pltpu.VMEM((NUM_TABLES, BPW, ED), jnp.float32),
            pltpu.SemaphoreType.DMA,
        ],
        compiler_params=pltpu.CompilerParams(
            use_tc_tiling_on_sc=False, needs_layout_passes=False
        ),
    )
    def k(bin_h, bout_h, pin_h, pout_h, proto_h, idx_h, out_h, idx_v,
          rows_v, sem):
        wid = lax.axis_index("s") * NC + lax.axis_index("c")
        base = wid * BPW
        pltpu.sync_copy(idx_h.at[pl.ds(wid * NIR, NIR)], idx_v)
        tables = (bin_h, bout_h, pin_h, pout_h, proto_h)
        copies = []
        for j, tab in enumerate(tables):
            for c in range(NCH):
                copies.append(
                    pltpu.async_copy(
                        tab.at[idx_v.at[j * NCH + c]],
                        rows_v.at[j, pl.ds(c * CHUNK, CHUNK)],
                        sem,
                    )
                )
        for cp in copies:
            cp.wait()
        for j in range(NUM_TABLES):
            pltpu.sync_copy(rows_v.at[j], out_h.at[j, pl.ds(base, BPW)])

    return k(bin_t, bout_t, pin_t, pout_t, proto_t, idx)


def _tc_mlp(xp, e, W1xp, W1ep, b1p, W2p, b2p, W3p, b3p):
    def body(xp_ref, e_ref, w1x_ref, w1e_ref, b1_ref, w2_ref, b2_ref, w3_ref,
             b3_ref, o_ref):
        h = jnp.dot(xp_ref[:], w1x_ref[:], preferred_element_type=jnp.float32)
        for j in range(NUM_TABLES):
            h = h + jnp.dot(e_ref[j], w1e_ref[j],
                            preferred_element_type=jnp.float32)
        h = jnp.maximum(h + b1_ref[:], 0.0)
        h = jnp.maximum(
            jnp.dot(h, w2_ref[:], preferred_element_type=jnp.float32) + b2_ref[:],
            0.0,
        )
        o_ref[:] = (
            jnp.dot(h, w3_ref[:], preferred_element_type=jnp.float32) + b3_ref[:]
        )

    return pl.pallas_call(
        body,
        grid=(BG // GBLK,),
        in_specs=[
            pl.BlockSpec((GBLK, G * 4), lambda i: (i, 0)),
            pl.BlockSpec((NUM_TABLES, GBLK, 128), lambda i: (0, i, 0)),
            pl.BlockSpec((G * 4, H8), lambda i: (0, 0)),
            pl.BlockSpec((NUM_TABLES, 128, H8), lambda i: (0, 0, 0)),
            pl.BlockSpec((1, H8), lambda i: (0, 0)),
            pl.BlockSpec((H8, H8), lambda i: (0, 0)),
            pl.BlockSpec((1, H8), lambda i: (0, 0)),
            pl.BlockSpec((H8, O8), lambda i: (0, 0)),
            pl.BlockSpec((1, O8), lambda i: (0, 0)),
        ],
        out_specs=pl.BlockSpec((GBLK, O8), lambda i: (i, 0)),
        out_shape=jax.ShapeDtypeStruct((BG, O8), jnp.float32),
    )(xp, e, W1xp, W1ep, b1p, W2p, b2p, W3p, b3p)


def kernel(x, bin_table, bout_table, pin_table, pout_table, proto_table,
           W1, b1, W2, b2, W3, b3):
    idx = (
        x[:, 4:9]
        .astype(jnp.int32)
        .T.reshape(NUM_TABLES, NW, NCH, CHUNK)
        .transpose(1, 0, 2, 3)
        .reshape(NW * NIR, CHUNK)
    )
    e = _sc_gather(bin_table, bout_table, pin_table, pout_table, proto_table,
                   idx)
    e = e.reshape(NUM_TABLES, BG, G * ED)
    xp = x[:, 0:4].reshape(BG, G * 4)
    eye8 = jnp.eye(G, dtype=jnp.float32)
    W1xp = jnp.kron(eye8, W1[0:4])
    W1ep = jax.vmap(lambda w: jnp.kron(eye8, w))(
        W1[4:].reshape(NUM_TABLES, ED, HIDDEN)
    )
    W2p = jnp.kron(eye8, W2)
    W3p = jnp.kron(eye8, W3)
    b1p = jnp.tile(b1, G).reshape(1, H8)
    b2p = jnp.tile(b2, G).reshape(1, H8)
    b3p = jnp.tile(b3, G).reshape(1, O8)
    o = _tc_mlp(xp, e, W1xp, W1ep, b1p, W2p, b2p, W3p, b3p)
    return o.reshape(B, OUT_DIM)
